# Initial kernel scaffold; baseline (speedup 1.0000x reference)
import jax
import jax.numpy as jnp
from jax.experimental import pallas as pl


def kernel(x, table):
    raise NotImplementedError("write your pallas kernel here")


# speedup vs baseline: 1.2312x; 1.2312x over previous
import functools
import math

import jax
import jax.numpy as jnp
from jax import lax
from jax.experimental import pallas as pl
from jax.experimental.pallas import tpu as pltpu
from jax.experimental.pallas import tpu_sc as plsc

D_MODEL = 128
SCALE = math.sqrt(float(D_MODEL))
LANES = 16
CHUNK = 128


def _sc_embed(table, idx2d):
    info = plsc.get_sparse_core_info()
    num_workers = info.num_cores * info.num_subcores
    n_chunks_total = idx2d.shape[0]
    chunks_per_w = n_chunks_total // num_workers
    b_per_w = chunks_per_w * CHUNK
    total = n_chunks_total * CHUNK
    mesh = plsc.VectorSubcoreMesh(core_axis_name="c", subcore_axis_name="s")

    @functools.partial(
        pl.kernel,
        mesh=mesh,
        out_type=jax.ShapeDtypeStruct((total, D_MODEL), jnp.float32),
        scratch_types=[
            pltpu.VMEM((chunks_per_w, CHUNK), jnp.int32),
            pltpu.VMEM((chunks_per_w, CHUNK, D_MODEL), jnp.float32),
            pltpu.SemaphoreType.DMA,
        ],
    )
    def k(table_hbm, idx_hbm, out_hbm, idx_v, rows_v, sem):
        wid = lax.axis_index("s") * info.num_cores + lax.axis_index("c")
        base_chunk = wid * chunks_per_w
        pltpu.sync_copy(idx_hbm.at[pl.ds(base_chunk, chunks_per_w)], idx_v)
        for c in range(chunks_per_w):
            pltpu.async_copy(table_hbm.at[idx_v.at[c]], rows_v.at[c], sem).wait()

            def scale_row(i, _):
                for j in range(D_MODEL // LANES):
                    sl = pl.ds(j * LANES, LANES)
                    rows_v[c, i, sl] = rows_v[c, i, sl] * SCALE
                return 0

            lax.fori_loop(0, CHUNK, scale_row, 0)
            pltpu.sync_copy(
                rows_v.at[c],
                out_hbm.at[pl.ds(wid * b_per_w + c * CHUNK, CHUNK)],
            )

    return k(table, idx2d)


def kernel(x, table):
    b, s = x.shape
    idx2d = x.reshape(-1, CHUNK).astype(jnp.int32)
    out = _sc_embed(table, idx2d)
    return out.reshape(b, s, D_MODEL)


# speedup vs baseline: 1.4159x; 1.1500x over previous
import functools
import math

import jax
import jax.numpy as jnp
from jax import lax
from jax.experimental import pallas as pl
from jax.experimental.pallas import tpu as pltpu
from jax.experimental.pallas import tpu_sc as plsc

D_MODEL = 128
SCALE = math.sqrt(float(D_MODEL))
LANES = 16
CHUNK = 128


def _sc_embed(table, idx2d):
    info = plsc.get_sparse_core_info()
    num_workers = info.num_cores * info.num_subcores
    n_chunks_total = idx2d.shape[0]
    chunks_per_w = n_chunks_total // num_workers
    b_per_w = chunks_per_w * CHUNK
    total = n_chunks_total * CHUNK
    mesh = plsc.VectorSubcoreMesh(core_axis_name="c", subcore_axis_name="s")

    @functools.partial(
        pl.kernel,
        mesh=mesh,
        out_type=jax.ShapeDtypeStruct((total, D_MODEL), jnp.float32),
        scratch_types=[
            pltpu.VMEM((chunks_per_w, CHUNK), jnp.int32),
            pltpu.VMEM((chunks_per_w, CHUNK, D_MODEL), jnp.float32),
            pltpu.SemaphoreType.DMA,
            pltpu.SemaphoreType.DMA,
        ],
    )
    def k(table_hbm, idx_hbm, out_hbm, idx_v, rows_v, gsem, osem):
        wid = lax.axis_index("s") * info.num_cores + lax.axis_index("c")
        base_chunk = wid * chunks_per_w
        pltpu.sync_copy(idx_hbm.at[pl.ds(base_chunk, chunks_per_w)], idx_v)
        gathers = [
            pltpu.async_copy(table_hbm.at[idx_v.at[c]], rows_v.at[c], gsem)
            for c in range(chunks_per_w)
        ]
        stores = []
        for c in range(chunks_per_w):
            gathers[c].wait()

            def scale_row(i, _):
                for j in range(D_MODEL // LANES):
                    sl = pl.ds(j * LANES, LANES)
                    rows_v[c, i, sl] = rows_v[c, i, sl] * SCALE
                return 0

            lax.fori_loop(0, CHUNK, scale_row, 0)
            stores.append(
                pltpu.async_copy(
                    rows_v.at[c],
                    out_hbm.at[pl.ds(wid * b_per_w + c * CHUNK, CHUNK)],
                    osem,
                )
            )
        for s in stores:
            s.wait()

    return k(table, idx2d)


def kernel(x, table):
    b, s = x.shape
    idx2d = x.reshape(-1, CHUNK).astype(jnp.int32)
    out = _sc_embed(table, idx2d)
    return out.reshape(b, s, D_MODEL)
